# baseline (speedup 1.0000x reference)
import functools

import jax
import jax.numpy as jnp
import numpy as np
from jax import lax
from jax.experimental import pallas as pl
from jax.experimental.pallas import tpu as pltpu
from jax.experimental.pallas import tpu_sc as plsc

_CONN = np.array(
    [[1, 2, 7, 6], [2, 3, 8, 7], [3, 4, 9, 8], [4, 5, 10, 9],
     [6, 7, 12, 11], [7, 8, 13, 12], [8, 9, 14, 13], [9, 10, 15, 14],
     [11, 12, 17, 16], [12, 13, 18, 17], [13, 14, 19, 18], [14, 15, 20, 19],
     [16, 17, 22, 21], [17, 18, 23, 22], [18, 19, 24, 23], [19, 20, 25, 24]],
    dtype=np.int32)

_N_PTS = 65536
_N_CELLS = 16
_N_NODES = 25
_NODE_PAD = 32

_NC, _NS, _L = 2, 16, 16
_NW = _NC * _NS
_CHUNK = _N_PTS // _NW
_GROUPS = _CHUNK // _L

_ITAB = jnp.asarray((_CONN.T - 1).reshape(-1), dtype=jnp.int32)
_TAB_WORDS = 4 * _N_CELLS + 3 * _NODE_PAD
_OFF_CX = 4 * _N_CELLS
_OFF_CY = _OFF_CX + _NODE_PAD
_OFF_VV = _OFF_CY + _NODE_PAD

_mesh = plsc.VectorSubcoreMesh(core_axis_name="c", subcore_axis_name="s")


@functools.partial(
    pl.kernel,
    mesh=_mesh,
    out_type=jax.ShapeDtypeStruct((_N_PTS,), jnp.float32),
    compiler_params=pltpu.CompilerParams(needs_layout_passes=False),
    scratch_types=[
        pltpu.VMEM((_CHUNK,), jnp.float32),
        pltpu.VMEM((_CHUNK,), jnp.float32),
        pltpu.VMEM((_CHUNK,), jnp.int32),
        pltpu.VMEM((_CHUNK,), jnp.float32),
        pltpu.VMEM((_TAB_WORDS,), jnp.int32),
        pltpu.VMEM((_N_CELLS,), jnp.float32),
        pltpu.VMEM((_N_CELLS,), jnp.float32),
        pltpu.VMEM((_N_CELLS,), jnp.float32),
        pltpu.VMEM((_N_CELLS,), jnp.float32),
        pltpu.VMEM((_N_CELLS,), jnp.float32),
        pltpu.VMEM((_N_CELLS,), jnp.float32),
        pltpu.SemaphoreType.DMA,
        pltpu.SemaphoreType.DMA,
        pltpu.SemaphoreType.DMA,
    ],
)
def _sc_interp(xt_hbm, cid_hbm, tab_hbm, out_hbm,
               x0_v, x1_v, cid_v, out_v, tab_v,
               c0_v, c1_v, c2_v, c3_v, c4_v, c5_v,
               sem0, sem1, sem2):
    wid = lax.axis_index("s") * _NC + lax.axis_index("c")
    base = wid * _CHUNK

    d0 = pltpu.async_copy(xt_hbm.at[0, pl.ds(base, _CHUNK)], x0_v, sem0)
    d0b = pltpu.async_copy(xt_hbm.at[1, pl.ds(base, _CHUNK)], x1_v, sem0)
    d1 = pltpu.async_copy(cid_hbm.at[pl.ds(base, _CHUNK)], cid_v, sem1)
    d2 = pltpu.async_copy(tab_hbm, tab_v, sem2)
    d2.wait()

    idx = [tab_v[pl.ds(k * _N_CELLS, _N_CELLS)] for k in range(4)]
    off_cx = jnp.full((_L,), _OFF_CX, jnp.int32)
    off_cy = jnp.full((_L,), _OFF_CY, jnp.int32)
    off_vv = jnp.full((_L,), _OFF_VV, jnp.int32)

    def _gf(i):
        return plsc.bitcast(plsc.load_gather(tab_v, [i]), jnp.float32)

    nx = [_gf(i + off_cx) for i in idx]
    ny = [_gf(i + off_cy) for i in idx]
    nv = [_gf(i + off_vv) for i in idx]

    c0 = jnp.zeros((_L,), jnp.float32)
    c1 = jnp.zeros((_L,), jnp.float32)
    c2 = jnp.zeros((_L,), jnp.float32)
    c3 = jnp.zeros((_L,), jnp.float32)
    c4 = jnp.zeros((_L,), jnp.float32)
    c5 = jnp.zeros((_L,), jnp.float32)
    for k in range(4):
        ax, ay = nx[k], ny[k]
        bx, by = nx[(k + 1) % 4], ny[(k + 1) % 4]
        ex, ey = nx[(k + 2) % 4], ny[(k + 2) % 4]
        dx, dy = nx[(k + 3) % 4], ny[(k + 3) % 4]
        bb1 = by - ey
        cc1 = ex - bx
        aa1 = -cc1 * by - bb1 * bx
        p12 = cc1 * (ay - by) + bb1 * (ax - bx)
        bb2 = dy - ey
        cc2 = ex - dx
        aa2 = -cc2 * dy - bb2 * dx
        p22 = cc2 * (ay - dy) + bb2 * (ax - dx)
        s = nv[k] / (p12 * p22)
        c0 = c0 + s * aa1 * aa2
        c1 = c1 + s * (aa1 * bb2 + bb1 * aa2)
        c2 = c2 + s * (aa1 * cc2 + cc1 * aa2)
        c3 = c3 + s * bb1 * bb2
        c4 = c4 + s * (bb1 * cc2 + cc1 * bb2)
        c5 = c5 + s * cc1 * cc2
    c0_v[...] = c0
    c1_v[...] = c1
    c2_v[...] = c2
    c3_v[...] = c3
    c4_v[...] = c4
    c5_v[...] = c5

    d0.wait()
    d0b.wait()
    d1.wait()

    @plsc.parallel_loop(0, _GROUPS, 1, unroll=8)
    def _(g):
        off = g * _L
        xx = x0_v[pl.ds(off, _L)]
        yy = x1_v[pl.ds(off, _L)]
        ci = cid_v[pl.ds(off, _L)]
        k0 = plsc.load_gather(c0_v, [ci])
        k1 = plsc.load_gather(c1_v, [ci])
        k2 = plsc.load_gather(c2_v, [ci])
        k3 = plsc.load_gather(c3_v, [ci])
        k4 = plsc.load_gather(c4_v, [ci])
        k5 = plsc.load_gather(c5_v, [ci])
        out_v[pl.ds(off, _L)] = (
            k0 + xx * (k1 + k3 * xx + k4 * yy) + yy * (k2 + k5 * yy))

    pltpu.sync_copy(out_v, out_hbm.at[pl.ds(base, _CHUNK)])


def kernel(x, cell_id, coordinates, nodal_values):
    xt = x.T
    cid = cell_id.astype(jnp.int32)
    coords = coordinates.reshape(-1, 2)
    ftab = jnp.zeros((3 * _NODE_PAD,), jnp.float32)
    ftab = ftab.at[:_N_NODES].set(coords[:, 0])
    ftab = ftab.at[_NODE_PAD:_NODE_PAD + _N_NODES].set(coords[:, 1])
    ftab = ftab.at[2 * _NODE_PAD:2 * _NODE_PAD + _N_NODES].set(
        nodal_values.reshape(-1))
    tab = jnp.concatenate([_ITAB, lax.bitcast_convert_type(ftab, jnp.int32)])
    return _sc_interp(xt, cid, tab)
